# baseline (speedup 1.0000x reference)
import functools

import jax
import jax.numpy as jnp
from jax import lax
from jax.experimental import pallas as pl
from jax.experimental.pallas import tpu as pltpu
from jax.experimental.pallas import tpu_sc as plsc

NUM_VIRTUAL_TOKENS = 50
TOKEN_DIM = 1024
EMB_DIM = 24 * 2 * TOKEN_DIM
BATCH = 32


def _body(prefix_hbm, emb_hbm, out_hbm, idx_v, buf0, buf1, gsem0, gsem1,
          ssem0, ssem1):
  c = lax.axis_index("c")
  s = lax.axis_index("s")
  w = s * 2 + c

  pltpu.sync_copy(prefix_hbm.at[w], idx_v)

  def g_start(i, buf, gsem):
    pltpu.async_copy(emb_hbm.at[idx_v.at[i]], buf, gsem)

  def g_wait(buf, gsem):
    pltpu.make_async_copy(emb_hbm.at[idx_v.at[0]], buf, gsem).wait()

  def s_start(i, buf, ssem):
    pltpu.async_copy(buf, out_hbm.at[w, pl.ds(i, 1)], ssem)

  def s_wait(i, buf, ssem):
    pltpu.make_async_copy(buf, out_hbm.at[w, pl.ds(i, 1)], ssem).wait()

  bufs = (buf0, buf1)
  gsems = (gsem0, gsem1)
  ssems = (ssem0, ssem1)

  g_start(0, buf0, gsem0)
  g_start(1, buf1, gsem1)

  def j_body(j, carry):
    for b in range(2):
      i = 2 * j + b
      g_wait(bufs[b], gsems[b])
      s_start(i, bufs[b], ssems[b])
      s_wait(i, bufs[b], ssems[b])

      @pl.when(i + 2 < NUM_VIRTUAL_TOKENS)
      def _():
        g_start(i + 2, bufs[b], gsems[b])

    return carry

  lax.fori_loop(0, NUM_VIRTUAL_TOKENS // 2, j_body, 0)


@jax.jit
def kernel(prefix, embedding):
  mesh = plsc.VectorSubcoreMesh(core_axis_name="c", subcore_axis_name="s")
  k = functools.partial(
      pl.kernel,
      out_type=jax.ShapeDtypeStruct((BATCH, NUM_VIRTUAL_TOKENS, EMB_DIM),
                                    jnp.float32),
      mesh=mesh,
      compiler_params=pltpu.CompilerParams(use_tc_tiling_on_sc=True),
      scratch_types=[
          pltpu.VMEM((NUM_VIRTUAL_TOKENS, 1), jnp.int32),
          pltpu.VMEM((1, EMB_DIM), jnp.float32),
          pltpu.VMEM((1, EMB_DIM), jnp.float32),
          pltpu.SemaphoreType.DMA,
          pltpu.SemaphoreType.DMA,
          pltpu.SemaphoreType.DMA,
          pltpu.SemaphoreType.DMA,
      ],
  )(_body)
  return k(prefix.reshape(BATCH, NUM_VIRTUAL_TOKENS, 1), embedding)


# speedup vs baseline: 1.8767x; 1.8767x over previous
import functools

import jax
import jax.numpy as jnp
from jax import lax
from jax.experimental import pallas as pl
from jax.experimental.pallas import tpu as pltpu
from jax.experimental.pallas import tpu_sc as plsc

NUM_VIRTUAL_TOKENS = 50
TOKEN_DIM = 1024
EMB_DIM = 24 * 2 * TOKEN_DIM
BATCH = 32


def _body(prefix_hbm, emb_hbm, out_hbm, idx_v, buf0, buf1, gsem0, gsem1,
          ssem0, ssem1):
  c = lax.axis_index("c")
  s = lax.axis_index("s")
  w = s * 2 + c

  pltpu.sync_copy(prefix_hbm.at[w], idx_v)

  def g_start(i, buf, gsem):
    pltpu.async_copy(emb_hbm.at[idx_v.at[i]], buf, gsem)

  def g_wait(buf, gsem):
    pltpu.make_async_copy(emb_hbm.at[idx_v.at[0]], buf, gsem).wait()

  def s_start(i, buf, ssem):
    pltpu.async_copy(buf, out_hbm.at[pl.ds(i, 1), w], ssem)

  def s_wait(i, buf, ssem):
    pltpu.make_async_copy(buf, out_hbm.at[pl.ds(i, 1), w], ssem).wait()

  bufs = (buf0, buf1)
  gsems = (gsem0, gsem1)
  ssems = (ssem0, ssem1)

  g_start(0, buf0, gsem0)
  g_start(1, buf1, gsem1)

  def j_body(j, carry):
    for b in range(2):
      i = 2 * j + b
      g_wait(bufs[b], gsems[b])
      s_start(i, bufs[b], ssems[b])
      s_wait(i, bufs[b], ssems[b])

      @pl.when(i + 2 < NUM_VIRTUAL_TOKENS)
      def _():
        g_start(i + 2, bufs[b], gsems[b])

    return carry

  lax.fori_loop(0, NUM_VIRTUAL_TOKENS // 2, j_body, 0)


@jax.jit
def kernel(prefix, embedding):
  mesh = plsc.VectorSubcoreMesh(core_axis_name="c", subcore_axis_name="s")
  k = functools.partial(
      pl.kernel,
      out_type=jax.ShapeDtypeStruct((NUM_VIRTUAL_TOKENS, BATCH, EMB_DIM),
                                    jnp.float32),
      mesh=mesh,
      compiler_params=pltpu.CompilerParams(use_tc_tiling_on_sc=True),
      scratch_types=[
          pltpu.VMEM((NUM_VIRTUAL_TOKENS, 1), jnp.int32),
          pltpu.VMEM((1, EMB_DIM), jnp.float32),
          pltpu.VMEM((1, EMB_DIM), jnp.float32),
          pltpu.SemaphoreType.DMA,
          pltpu.SemaphoreType.DMA,
          pltpu.SemaphoreType.DMA,
          pltpu.SemaphoreType.DMA,
      ],
  )(_body)
  out = k(prefix.reshape(BATCH, NUM_VIRTUAL_TOKENS, 1), embedding)
  return out.transpose(1, 0, 2)


# speedup vs baseline: 1.8793x; 1.0014x over previous
import functools

import jax
import jax.numpy as jnp
from jax import lax
from jax.experimental import pallas as pl
from jax.experimental.pallas import tpu as pltpu
from jax.experimental.pallas import tpu_sc as plsc

NUM_VIRTUAL_TOKENS = 50
TOKEN_DIM = 1024
EMB_DIM = 24 * 2 * TOKEN_DIM
BATCH = 32


def _body(prefix_hbm, emb_hbm, out_hbm, idx_v, buf0, buf1, gsem0, gsem1,
          ssem0, ssem1):
  c = lax.axis_index("c")
  s = lax.axis_index("s")
  w = s * 2 + c

  pltpu.sync_copy(prefix_hbm.at[pl.ds(w, 1)], idx_v)

  def g_start(i, buf, gsem):
    pltpu.async_copy(emb_hbm.at[idx_v.at[0, pl.ds(i, 1)]], buf, gsem)

  def g_wait(buf, gsem):
    pltpu.make_async_copy(emb_hbm.at[idx_v.at[0, pl.ds(0, 1)]], buf,
                          gsem).wait()

  def s_start(i, buf, ssem):
    pltpu.async_copy(buf, out_hbm.at[pl.ds(i, 1), w], ssem)

  def s_wait(i, buf, ssem):
    pltpu.make_async_copy(buf, out_hbm.at[pl.ds(i, 1), w], ssem).wait()

  bufs = (buf0, buf1)
  gsems = (gsem0, gsem1)
  ssems = (ssem0, ssem1)

  g_start(0, buf0, gsem0)
  g_start(1, buf1, gsem1)

  def j_body(j, carry):
    for b in range(2):
      i = 2 * j + b
      g_wait(bufs[b], gsems[b])
      s_start(i, bufs[b], ssems[b])
      s_wait(i, bufs[b], ssems[b])

      @pl.when(i + 2 < NUM_VIRTUAL_TOKENS)
      def _():
        g_start(i + 2, bufs[b], gsems[b])

    return carry

  lax.fori_loop(0, NUM_VIRTUAL_TOKENS // 2, j_body, 0)


@jax.jit
def kernel(prefix, embedding):
  mesh = plsc.VectorSubcoreMesh(core_axis_name="c", subcore_axis_name="s")
  k = functools.partial(
      pl.kernel,
      out_type=jax.ShapeDtypeStruct((NUM_VIRTUAL_TOKENS, BATCH, EMB_DIM),
                                    jnp.float32),
      mesh=mesh,
      compiler_params=pltpu.CompilerParams(use_tc_tiling_on_sc=True),
      scratch_types=[
          pltpu.VMEM((1, NUM_VIRTUAL_TOKENS), jnp.int32),
          pltpu.VMEM((1, EMB_DIM), jnp.float32),
          pltpu.VMEM((1, EMB_DIM), jnp.float32),
          pltpu.SemaphoreType.DMA,
          pltpu.SemaphoreType.DMA,
          pltpu.SemaphoreType.DMA,
          pltpu.SemaphoreType.DMA,
      ],
  )(_body)
  out = k(prefix, embedding)
  return out.transpose(1, 0, 2)
